# speedup vs baseline: 1.1623x; 1.1623x over previous
import functools

import jax
import jax.numpy as jnp
from jax import lax
from jax.experimental import pallas as pl
from jax.experimental.pallas import tpu as pltpu
from jax.experimental.pallas import tpu_sc as plsc

_NUM_CORES = 2
_NUM_SUBCORES = 16
_LANES = 16


def kernel(x, tkn_table, pos_table):
    B, S = x.shape
    V, D = tkn_table.shape
    NW = _NUM_CORES * _NUM_SUBCORES
    C = S // NW
    assert S % NW == 0 and D % _LANES == 0

    x = x.astype(jnp.int32)

    mesh = plsc.VectorSubcoreMesh(core_axis_name="c", subcore_axis_name="s")

    @functools.partial(
        pl.kernel,
        mesh=mesh,
        out_type=jax.ShapeDtypeStruct((B, S, D), jnp.float32),
        scratch_types=[
            pltpu.VMEM((C,), jnp.int32),
            pltpu.VMEM((C, D), jnp.float32),
            pltpu.VMEM((C, D), jnp.float32),
            pltpu.SemaphoreType.DMA,
        ],
    )
    def emb(x_hbm, tkn_hbm, pos_hbm, out_hbm, idx_v, pos_v, tkn_v, sem):
        wid = lax.axis_index("s") * _NUM_CORES + lax.axis_index("c")
        s0 = wid * C
        pltpu.sync_copy(pos_hbm.at[pl.ds(s0, C)], pos_v)
        for b in range(B):
            pltpu.sync_copy(x_hbm.at[b, pl.ds(s0, C)], idx_v)
            pltpu.async_copy(tkn_hbm.at[idx_v], tkn_v, sem).wait()

            def row_body(r, carry):
                for c in range(D // _LANES):
                    sl = pl.ds(c * _LANES, _LANES)
                    tkn_v[r, sl] = tkn_v[r, sl] + pos_v[r, sl]
                return carry

            lax.fori_loop(0, C, row_body, 0)
            pltpu.sync_copy(tkn_v, out_hbm.at[b, pl.ds(s0, C)])

    return emb(x, tkn_table, pos_table)
